# baseline (speedup 1.0000x reference)
import functools

import jax
import jax.numpy as jnp
from jax import lax
from jax.experimental import pallas as pl
from jax.experimental.pallas import tpu as pltpu
from jax.experimental.pallas import tpu_sc as plsc

B, F, V, D = 16384, 26, 100000, 64
T = 2
FD = F * D
IN_DIM = 2 * FD
BF = B * F
DW = 2 * D

NC, NS = 2, 16
NW = NC * NS
CH = 128
NCHUNK = BF // CH // NW

TB = 512


def _sc_gather(idx, tab_a, tab_b):
    mesh = plsc.VectorSubcoreMesh(core_axis_name="c", subcore_axis_name="s")
    out_t = [jax.ShapeDtypeStruct((F, B, DW), jnp.bfloat16)] * 2

    @functools.partial(
        pl.kernel,
        mesh=mesh,
        out_type=out_t,
        scratch_types=[
            pltpu.VMEM((CH,), jnp.int32),
            pltpu.VMEM((CH, DW), jnp.bfloat16),
            pltpu.VMEM((CH, DW), jnp.bfloat16),
            pltpu.SemaphoreType.DMA,
        ],
        compiler_params=pltpu.CompilerParams(use_tc_tiling_on_sc=False),
    )
    def k(idx_hbm, a_hbm, b_hbm, oa_hbm, ob_hbm, i0_v, ra, rb, sem):
        wid = lax.axis_index("s") * NC + lax.axis_index("c")
        chunks_per_plane = B // CH

        @pl.loop(0, NCHUNK)
        def _(c):
            cg = wid * NCHUNK + c
            fi = cg // chunks_per_plane
            bs = (cg % chunks_per_plane) * CH
            pltpu.sync_copy(idx_hbm.at[pl.ds(cg * CH, CH)], i0_v)
            ca = pltpu.async_copy(a_hbm.at[i0_v], ra, sem)
            cb = pltpu.async_copy(b_hbm.at[i0_v], rb, sem)
            ca.wait()
            cb.wait()
            pltpu.sync_copy(ra, oa_hbm.at[fi].at[pl.ds(bs, CH)])
            pltpu.sync_copy(rb, ob_hbm.at[fi].at[pl.ds(bs, CH)])

    return k(idx, tab_a, tab_b)


def _dense_body(ga_ref, gb_ref, w1a_ref, w1b_ref, b1_ref, w2_ref, b2_ref,
                w3_ref, b3_ref, tw1_ref, tb1_ref, tw2_ref, tb2_ref, out_ref):
    logits = []
    for i in range(T):
        g_ref = ga_ref if i == 0 else gb_ref
        w1_ref = w1a_ref if i == 0 else w1b_ref
        cc = jnp.concatenate([g_ref[f] for f in range(F)], axis=1)
        h = jnp.maximum(jnp.dot(cc, w1_ref[...], preferred_element_type=jnp.float32) + b1_ref[...], 0.0)
        h = jnp.maximum(jnp.dot(h.astype(jnp.bfloat16), w2_ref[...], preferred_element_type=jnp.float32) + b2_ref[...], 0.0)
        h = jnp.maximum(jnp.dot(h.astype(jnp.bfloat16), w3_ref[...], preferred_element_type=jnp.float32) + b3_ref[...], 0.0)
        t = jnp.maximum(jnp.dot(h.astype(jnp.bfloat16), tw1_ref[i], preferred_element_type=jnp.float32) + tb1_ref[i:i + 1, :], 0.0)
        logit = jnp.sum(t * tw2_ref[i:i + 1, :], axis=1, keepdims=True) + tb2_ref[i:i + 1, :]
        logits.append(logit)
    out_ref[...] = jax.nn.sigmoid(jnp.concatenate(logits, axis=1))


def _tc_dense(ga, gb, W1a, W1b, b1, W2, b2, W3, b3, tw1, tb1, tw2r, tb2):
    full = lambda shape: pl.BlockSpec(shape, lambda i: (0,) * len(shape))
    return pl.pallas_call(
        _dense_body,
        grid=(B // TB,),
        in_specs=[
            pl.BlockSpec((F, TB, DW), lambda i: (0, i, 0)),
            pl.BlockSpec((F, TB, DW), lambda i: (0, i, 0)),
            full((IN_DIM, 256)),
            full((IN_DIM, 256)),
            full((1, 256)),
            full((256, 128)),
            full((1, 128)),
            full((128, 64)),
            full((1, 64)),
            full((T, 64, 64)),
            full((T, 64)),
            full((T, 64)),
            full((T, 1)),
        ],
        out_specs=pl.BlockSpec((TB, T), lambda i: (i, 0)),
        out_shape=jax.ShapeDtypeStruct((B, T), jnp.float32),
    )(ga, gb, W1a, W1b, b1, W2, b2, W3, b3, tw1, tb1, tw2r, tb2)


def kernel(x, shared_table, task_tables, W1, b1, W2, b2, W3, b3, tw1, tb1, tw2, tb2):
    bf = jnp.bfloat16
    idx = x.T.reshape(BF)
    tab_a = jnp.concatenate([shared_table, task_tables[0]], axis=1).astype(bf)
    tab_b = jnp.concatenate([task_tables[1], shared_table], axis=1).astype(bf)
    w14 = W1.reshape(2, F, D, 256)
    W1a = w14.transpose(1, 0, 2, 3).reshape(IN_DIM, 256).astype(bf)
    W1b = w14[::-1].transpose(1, 0, 2, 3).reshape(IN_DIM, 256).astype(bf)
    ga, gb = _sc_gather(idx, tab_a, tab_b)
    return _tc_dense(
        ga, gb, W1a, W1b,
        b1.reshape(1, 256), W2.astype(bf), b2.reshape(1, 128),
        W3.astype(bf), b3.reshape(1, 64),
        tw1.astype(bf), tb1, tw2.reshape(T, 64), tb2,
    )


# speedup vs baseline: 2.0484x; 2.0484x over previous
import functools

import jax
import jax.numpy as jnp
from jax import lax
from jax.experimental import pallas as pl
from jax.experimental.pallas import tpu as pltpu
from jax.experimental.pallas import tpu_sc as plsc

B, F, V, D = 16384, 26, 100000, 64
T = 2
FD = F * D
IN_DIM = 2 * FD
BF = B * F
DW = 2 * D

NC, NS = 2, 16
NW = NC * NS
CH = 128
NCHUNK = BF // CH // NW

TB = 512


def _sc_gather(idx, tab_a, tab_b):
    mesh = plsc.VectorSubcoreMesh(core_axis_name="c", subcore_axis_name="s")
    out_t = [jax.ShapeDtypeStruct((F, B, DW), jnp.float32)] * 2

    @functools.partial(
        pl.kernel,
        mesh=mesh,
        out_type=out_t,
        scratch_types=[
            pltpu.VMEM((CH,), jnp.int32),
            pltpu.VMEM((CH, DW), jnp.float32),
            pltpu.VMEM((CH, DW), jnp.float32),
            pltpu.SemaphoreType.DMA,
        ],
        compiler_params=pltpu.CompilerParams(use_tc_tiling_on_sc=True),
    )
    def k(idx_hbm, a_hbm, b_hbm, oa_hbm, ob_hbm, i0_v, ra, rb, sem):
        wid = lax.axis_index("s") * NC + lax.axis_index("c")
        chunks_per_plane = B // CH

        @pl.loop(0, NCHUNK)
        def _(c):
            cg = wid * NCHUNK + c
            fi = cg // chunks_per_plane
            bs = (cg % chunks_per_plane) * CH
            pltpu.sync_copy(idx_hbm.at[pl.ds(cg * CH, CH)], i0_v)
            ca = pltpu.async_copy(a_hbm.at[i0_v], ra, sem)
            cb = pltpu.async_copy(b_hbm.at[i0_v], rb, sem)
            ca.wait()
            cb.wait()
            pltpu.sync_copy(ra, oa_hbm.at[fi].at[pl.ds(bs, CH)])
            pltpu.sync_copy(rb, ob_hbm.at[fi].at[pl.ds(bs, CH)])

    return k(idx, tab_a, tab_b)


def _dense_body(ga_ref, gb_ref, w1_ref, b1_ref, w2_ref, b2_ref,
                w3_ref, b3_ref, tw1_ref, tb1_ref, tw2_ref, tb2_ref, out_ref):
    logits = []
    for i in range(T):
        g_ref = ga_ref if i == 0 else gb_ref
        cc = jnp.concatenate([g_ref[f] for f in range(F)], axis=1).astype(jnp.bfloat16)
        h = jnp.maximum(jnp.dot(cc, w1_ref[...], preferred_element_type=jnp.float32) + b1_ref[...], 0.0)
        h = jnp.maximum(jnp.dot(h.astype(jnp.bfloat16), w2_ref[...], preferred_element_type=jnp.float32) + b2_ref[...], 0.0)
        h = jnp.maximum(jnp.dot(h.astype(jnp.bfloat16), w3_ref[...], preferred_element_type=jnp.float32) + b3_ref[...], 0.0)
        t = jnp.maximum(jnp.dot(h.astype(jnp.bfloat16), tw1_ref[i], preferred_element_type=jnp.float32) + tb1_ref[i:i + 1, :], 0.0)
        logit = jnp.sum(t * tw2_ref[i:i + 1, :], axis=1, keepdims=True) + tb2_ref[i:i + 1, :]
        logits.append(logit)
    out_ref[...] = jax.nn.sigmoid(jnp.concatenate(logits, axis=1))


def _tc_dense(ga, gb, W1p, b1, W2, b2, W3, b3, tw1, tb1, tw2r, tb2):
    full = lambda shape: pl.BlockSpec(shape, lambda i: (0,) * len(shape))
    return pl.pallas_call(
        _dense_body,
        grid=(B // TB,),
        in_specs=[
            pl.BlockSpec((F, TB, DW), lambda i: (0, i, 0)),
            pl.BlockSpec((F, TB, DW), lambda i: (0, i, 0)),
            full((IN_DIM, 256)),
            full((1, 256)),
            full((256, 128)),
            full((1, 128)),
            full((128, 64)),
            full((1, 64)),
            full((T, 64, 64)),
            full((T, 64)),
            full((T, 64)),
            full((T, 1)),
        ],
        out_specs=pl.BlockSpec((TB, T), lambda i: (i, 0)),
        out_shape=jax.ShapeDtypeStruct((B, T), jnp.float32),
    )(ga, gb, W1p, b1, W2, b2, W3, b3, tw1, tb1, tw2r, tb2)


def kernel(x, shared_table, task_tables, W1, b1, W2, b2, W3, b3, tw1, tb1, tw2, tb2):
    bf = jnp.bfloat16
    idx = x.T.reshape(BF)
    tab_a = jnp.concatenate([shared_table, task_tables[0]], axis=1)
    tab_b = jnp.concatenate([shared_table, task_tables[1]], axis=1)
    W1p = W1.reshape(2, F, D, 256).transpose(1, 0, 2, 3).reshape(IN_DIM, 256).astype(bf)
    ga, gb = _sc_gather(idx, tab_a, tab_b)
    return _tc_dense(
        ga, gb, W1p,
        b1.reshape(1, 256), W2.astype(bf), b2.reshape(1, 128),
        W3.astype(bf), b3.reshape(1, 64),
        tw1.astype(bf), tb1, tw2.reshape(T, 64), tb2,
    )


# speedup vs baseline: 2.7425x; 1.3388x over previous
import functools

import jax
import jax.numpy as jnp
from jax import lax
from jax.experimental import pallas as pl
from jax.experimental.pallas import tpu as pltpu
from jax.experimental.pallas import tpu_sc as plsc

B, F, V, D = 16384, 26, 100000, 64
T = 2
FD = F * D
IN_DIM = 2 * FD
BF = B * F
DW = 2 * D

NC, NS = 2, 16
NW = NC * NS
CH = 128
NCHUNK = BF // CH // NW

TB = 512


def _sc_gather(idx, tabp):
    mesh = plsc.VectorSubcoreMesh(core_axis_name="c", subcore_axis_name="s")
    out_t = jax.ShapeDtypeStruct((F, B, DW), jnp.int32)

    @functools.partial(
        pl.kernel,
        mesh=mesh,
        out_type=out_t,
        scratch_types=[
            pltpu.VMEM((CH,), jnp.int32),
            pltpu.VMEM((CH, DW), jnp.int32),
            pltpu.SemaphoreType.DMA,
        ],
        compiler_params=pltpu.CompilerParams(use_tc_tiling_on_sc=True),
    )
    def k(idx_hbm, tab_hbm, out_hbm, i0_v, rows_v, sem):
        wid = lax.axis_index("s") * NC + lax.axis_index("c")
        chunks_per_plane = B // CH

        @pl.loop(0, NCHUNK)
        def _(c):
            cg = wid * NCHUNK + c
            fi = cg // chunks_per_plane
            bs = (cg % chunks_per_plane) * CH
            pltpu.sync_copy(idx_hbm.at[pl.ds(cg * CH, CH)], i0_v)
            pltpu.async_copy(tab_hbm.at[i0_v], rows_v, sem).wait()
            pltpu.sync_copy(rows_v, out_hbm.at[fi].at[pl.ds(bs, CH)])

    return k(idx, tabp)


def _dense_body(g_ref, w1_ref, b1_ref, w2_ref, b2_ref,
                w3_ref, b3_ref, tw1_ref, tb1_ref, tw2_ref, tb2_ref, out_ref):
    logits = []
    for i in range(T):
        planes = []
        for f in range(F):
            v = g_ref[f]
            if i == 0:
                v = v << 16
            else:
                v = v & jnp.int32(-65536)
            planes.append(lax.bitcast_convert_type(v, jnp.float32))
        cc = jnp.concatenate(planes, axis=1).astype(jnp.bfloat16)
        h = jnp.maximum(jnp.dot(cc, w1_ref[...], preferred_element_type=jnp.float32) + b1_ref[...], 0.0)
        h = jnp.maximum(jnp.dot(h.astype(jnp.bfloat16), w2_ref[...], preferred_element_type=jnp.float32) + b2_ref[...], 0.0)
        h = jnp.maximum(jnp.dot(h.astype(jnp.bfloat16), w3_ref[...], preferred_element_type=jnp.float32) + b3_ref[...], 0.0)
        t = jnp.maximum(jnp.dot(h.astype(jnp.bfloat16), tw1_ref[i], preferred_element_type=jnp.float32) + tb1_ref[i:i + 1, :], 0.0)
        logit = jnp.sum(t * tw2_ref[i:i + 1, :], axis=1, keepdims=True) + tb2_ref[i:i + 1, :]
        logits.append(logit)
    out_ref[...] = jax.nn.sigmoid(jnp.concatenate(logits, axis=1))


def _tc_dense(g, W1p, b1, W2, b2, W3, b3, tw1, tb1, tw2r, tb2):
    full = lambda shape: pl.BlockSpec(shape, lambda i: (0,) * len(shape))
    return pl.pallas_call(
        _dense_body,
        grid=(B // TB,),
        in_specs=[
            pl.BlockSpec((F, TB, DW), lambda i: (0, i, 0)),
            full((IN_DIM, 256)),
            full((1, 256)),
            full((256, 128)),
            full((1, 128)),
            full((128, 64)),
            full((1, 64)),
            full((T, 64, 64)),
            full((T, 64)),
            full((T, 64)),
            full((T, 1)),
        ],
        out_specs=pl.BlockSpec((TB, T), lambda i: (i, 0)),
        out_shape=jax.ShapeDtypeStruct((B, T), jnp.float32),
    )(g, W1p, b1, W2, b2, W3, b3, tw1, tb1, tw2r, tb2)


def kernel(x, shared_table, task_tables, W1, b1, W2, b2, W3, b3, tw1, tb1, tw2, tb2):
    bf = jnp.bfloat16
    idx = x.T.reshape(BF)
    tab_a = jnp.concatenate([shared_table, task_tables[0]], axis=1).astype(bf)
    tab_b = jnp.concatenate([shared_table, task_tables[1]], axis=1).astype(bf)
    tabp = lax.bitcast_convert_type(jnp.stack([tab_a, tab_b], axis=-1), jnp.int32)
    W1p = W1.reshape(2, F, D, 256).transpose(1, 0, 2, 3).reshape(IN_DIM, 256).astype(bf)
    g = _sc_gather(idx, tabp)
    return _tc_dense(
        g, W1p,
        b1.reshape(1, 256), W2.astype(bf), b2.reshape(1, 128),
        W3.astype(bf), b3.reshape(1, 64),
        tw1.astype(bf), tb1, tw2.reshape(T, 64), tb2,
    )


# speedup vs baseline: 2.7453x; 1.0010x over previous
import functools

import jax
import jax.numpy as jnp
from jax import lax
from jax.experimental import pallas as pl
from jax.experimental.pallas import tpu as pltpu
from jax.experimental.pallas import tpu_sc as plsc

B, F, V, D = 16384, 26, 100000, 64
T = 2
FD = F * D
IN_DIM = 2 * FD
BF = B * F
DW = 2 * D

NC, NS = 2, 16
NW = NC * NS
CH = 128
NCHUNK = BF // CH // NW

TB = 512


def _sc_gather(idx, tabp):
    mesh = plsc.VectorSubcoreMesh(core_axis_name="c", subcore_axis_name="s")
    out_t = jax.ShapeDtypeStruct((F, B, DW), jnp.int32)

    @functools.partial(
        pl.kernel,
        mesh=mesh,
        out_type=out_t,
        scratch_types=[
            pltpu.VMEM((CH,), jnp.int32),
            pltpu.VMEM((CH, DW), jnp.int32),
            pltpu.SemaphoreType.DMA,
        ],
        compiler_params=pltpu.CompilerParams(use_tc_tiling_on_sc=True),
    )
    def k(idx_hbm, tab_hbm, out_hbm, i0_v, rows_v, sem):
        wid = lax.axis_index("s") * NC + lax.axis_index("c")
        chunks_per_plane = B // CH

        @pl.loop(0, NCHUNK)
        def _(c):
            cg = wid * NCHUNK + c
            fi = cg // chunks_per_plane
            bs = (cg % chunks_per_plane) * CH
            pltpu.sync_copy(idx_hbm.at[pl.ds(cg * CH, CH)], i0_v)
            pltpu.async_copy(tab_hbm.at[i0_v], rows_v, sem).wait()
            pltpu.sync_copy(rows_v, out_hbm.at[fi].at[pl.ds(bs, CH)])

    return k(idx, tabp)


def _dense_body(g_ref, w1_ref, b1_ref, w2_ref, b2_ref,
                w3_ref, b3_ref, tw1_ref, tb1_ref, tw2_ref, tb2_ref, out_ref):
    gcat = jnp.concatenate([g_ref[f] for f in range(F)], axis=1)
    logits = []
    for i in range(T):
        v = (gcat << 16) if i == 0 else (gcat & jnp.int32(-65536))
        cc = lax.bitcast_convert_type(v, jnp.float32).astype(jnp.bfloat16)
        h = jnp.maximum(jnp.dot(cc, w1_ref[...], preferred_element_type=jnp.float32) + b1_ref[...], 0.0)
        h = jnp.maximum(jnp.dot(h.astype(jnp.bfloat16), w2_ref[...], preferred_element_type=jnp.float32) + b2_ref[...], 0.0)
        h = jnp.maximum(jnp.dot(h.astype(jnp.bfloat16), w3_ref[...], preferred_element_type=jnp.float32) + b3_ref[...], 0.0)
        t = jnp.maximum(jnp.dot(h.astype(jnp.bfloat16), tw1_ref[i], preferred_element_type=jnp.float32) + tb1_ref[i:i + 1, :], 0.0)
        logit = jnp.sum(t * tw2_ref[i:i + 1, :], axis=1, keepdims=True) + tb2_ref[i:i + 1, :]
        logits.append(logit)
    out_ref[...] = jax.nn.sigmoid(jnp.concatenate(logits, axis=1))


def _tc_dense(g, W1p, b1, W2, b2, W3, b3, tw1, tb1, tw2r, tb2):
    full = lambda shape: pl.BlockSpec(shape, lambda i: (0,) * len(shape))
    return pl.pallas_call(
        _dense_body,
        grid=(B // TB,),
        in_specs=[
            pl.BlockSpec((F, TB, DW), lambda i: (0, i, 0)),
            full((IN_DIM, 256)),
            full((1, 256)),
            full((256, 128)),
            full((1, 128)),
            full((128, 64)),
            full((1, 64)),
            full((T, 64, 64)),
            full((T, 64)),
            full((T, 64)),
            full((T, 1)),
        ],
        out_specs=pl.BlockSpec((TB, T), lambda i: (i, 0)),
        out_shape=jax.ShapeDtypeStruct((B, T), jnp.float32),
    )(g, W1p, b1, W2, b2, W3, b3, tw1, tb1, tw2r, tb2)


def kernel(x, shared_table, task_tables, W1, b1, W2, b2, W3, b3, tw1, tb1, tw2, tb2):
    bf = jnp.bfloat16
    idx = x.T.reshape(BF)
    tab_a = jnp.concatenate([shared_table, task_tables[0]], axis=1).astype(bf)
    tab_b = jnp.concatenate([shared_table, task_tables[1]], axis=1).astype(bf)
    tabp = lax.bitcast_convert_type(jnp.stack([tab_a, tab_b], axis=-1), jnp.int32)
    W1p = W1.reshape(2, F, D, 256).transpose(1, 0, 2, 3).reshape(IN_DIM, 256).astype(bf)
    g = _sc_gather(idx, tabp)
    return _tc_dense(
        g, W1p,
        b1.reshape(1, 256), W2.astype(bf), b2.reshape(1, 128),
        W3.astype(bf), b3.reshape(1, 64),
        tw1.astype(bf), tb1, tw2.reshape(T, 64), tb2,
    )


# speedup vs baseline: 3.3411x; 1.2170x over previous
import functools

import jax
import jax.numpy as jnp
from jax import lax
from jax.experimental import pallas as pl
from jax.experimental.pallas import tpu as pltpu
from jax.experimental.pallas import tpu_sc as plsc

B, F, V, D = 16384, 26, 100000, 64
T = 2
FD = F * D
IN_DIM = 2 * FD
BF = B * F
DW = 2 * D

NC, NS = 2, 16
NW = NC * NS
CH = 128
NCHUNK = BF // CH // NW

TB = 512


def _sc_gather(idx, tabp):
    mesh = plsc.VectorSubcoreMesh(core_axis_name="c", subcore_axis_name="s")
    out_t = jax.ShapeDtypeStruct((F, B, DW), jnp.int32)

    @functools.partial(
        pl.kernel,
        mesh=mesh,
        out_type=out_t,
        scratch_types=[
            pltpu.VMEM((CH,), jnp.int32),
            pltpu.VMEM((CH,), jnp.int32),
            pltpu.VMEM((CH, DW), jnp.int32),
            pltpu.VMEM((CH, DW), jnp.int32),
            pltpu.SemaphoreType.DMA,
            pltpu.SemaphoreType.DMA,
        ],
        compiler_params=pltpu.CompilerParams(use_tc_tiling_on_sc=True),
    )
    def k(idx_hbm, tab_hbm, out_hbm, i0_v, i1_v, r0, r1, sem0, sem1):
        wid = lax.axis_index("s") * NC + lax.axis_index("c")
        chunks_per_plane = B // CH
        base = wid * NCHUNK

        def out_slice(cg):
            fi = cg // chunks_per_plane
            bs = (cg % chunks_per_plane) * CH
            return out_hbm.at[fi].at[pl.ds(bs, CH)]

        pltpu.sync_copy(idx_hbm.at[pl.ds(base * CH, CH)], i0_v)
        pltpu.async_copy(tab_hbm.at[i0_v], r0, sem0)

        @pl.loop(0, NCHUNK // 2)
        def _(j):
            c0 = base + 2 * j
            pltpu.sync_copy(idx_hbm.at[pl.ds((c0 + 1) * CH, CH)], i1_v)
            pltpu.async_copy(tab_hbm.at[i1_v], r1, sem1)
            pltpu.make_async_copy(tab_hbm.at[i0_v], r0, sem0).wait()
            pltpu.sync_copy(r0, out_slice(c0))

            @pl.when(j < NCHUNK // 2 - 1)
            def _():
                pltpu.sync_copy(idx_hbm.at[pl.ds((c0 + 2) * CH, CH)], i0_v)
                pltpu.async_copy(tab_hbm.at[i0_v], r0, sem0)

            pltpu.make_async_copy(tab_hbm.at[i1_v], r1, sem1).wait()
            pltpu.sync_copy(r1, out_slice(c0 + 1))

    return k(idx, tabp)


def _dense_body(g_ref, w1_ref, b1_ref, w2_ref, b2_ref,
                w3_ref, b3_ref, tw1_ref, tb1_ref, tw2_ref, tb2_ref, out_ref):
    gcat = jnp.concatenate([g_ref[f] for f in range(F)], axis=1)
    logits = []
    for i in range(T):
        v = (gcat << 16) if i == 0 else (gcat & jnp.int32(-65536))
        cc = lax.bitcast_convert_type(v, jnp.float32).astype(jnp.bfloat16)
        h = jnp.maximum(jnp.dot(cc, w1_ref[...], preferred_element_type=jnp.float32) + b1_ref[...], 0.0)
        h = jnp.maximum(jnp.dot(h.astype(jnp.bfloat16), w2_ref[...], preferred_element_type=jnp.float32) + b2_ref[...], 0.0)
        h = jnp.maximum(jnp.dot(h.astype(jnp.bfloat16), w3_ref[...], preferred_element_type=jnp.float32) + b3_ref[...], 0.0)
        t = jnp.maximum(jnp.dot(h.astype(jnp.bfloat16), tw1_ref[i], preferred_element_type=jnp.float32) + tb1_ref[i:i + 1, :], 0.0)
        logit = jnp.sum(t * tw2_ref[i:i + 1, :], axis=1, keepdims=True) + tb2_ref[i:i + 1, :]
        logits.append(logit)
    out_ref[...] = jax.nn.sigmoid(jnp.concatenate(logits, axis=1))


def _tc_dense(g, W1p, b1, W2, b2, W3, b3, tw1, tb1, tw2r, tb2):
    full = lambda shape: pl.BlockSpec(shape, lambda i: (0,) * len(shape))
    return pl.pallas_call(
        _dense_body,
        grid=(B // TB,),
        in_specs=[
            pl.BlockSpec((F, TB, DW), lambda i: (0, i, 0)),
            full((IN_DIM, 256)),
            full((1, 256)),
            full((256, 128)),
            full((1, 128)),
            full((128, 64)),
            full((1, 64)),
            full((T, 64, 64)),
            full((T, 64)),
            full((T, 64)),
            full((T, 1)),
        ],
        out_specs=pl.BlockSpec((TB, T), lambda i: (i, 0)),
        out_shape=jax.ShapeDtypeStruct((B, T), jnp.float32),
    )(g, W1p, b1, W2, b2, W3, b3, tw1, tb1, tw2r, tb2)


def kernel(x, shared_table, task_tables, W1, b1, W2, b2, W3, b3, tw1, tb1, tw2, tb2):
    bf = jnp.bfloat16
    idx = x.T.reshape(BF)
    tab_a = jnp.concatenate([shared_table, task_tables[0]], axis=1).astype(bf)
    tab_b = jnp.concatenate([shared_table, task_tables[1]], axis=1).astype(bf)
    tabp = lax.bitcast_convert_type(jnp.stack([tab_a, tab_b], axis=-1), jnp.int32)
    W1p = W1.reshape(2, F, D, 256).transpose(1, 0, 2, 3).reshape(IN_DIM, 256).astype(bf)
    g = _sc_gather(idx, tabp)
    return _tc_dense(
        g, W1p,
        b1.reshape(1, 256), W2.astype(bf), b2.reshape(1, 128),
        W3.astype(bf), b3.reshape(1, 64),
        tw1.astype(bf), tb1, tw2.reshape(T, 64), tb2,
    )


# speedup vs baseline: 3.4080x; 1.0200x over previous
import functools

import jax
import jax.numpy as jnp
from jax import lax
from jax.experimental import pallas as pl
from jax.experimental.pallas import tpu as pltpu
from jax.experimental.pallas import tpu_sc as plsc

B, F, V, D = 16384, 26, 100000, 64
T = 2
FD = F * D
IN_DIM = 2 * FD
BF = B * F
DW = 2 * D

NC, NS = 2, 16
NW = NC * NS
CH = 128
NCHUNK = BF // CH // NW

TB = 1024


def _sc_gather(idx, tabp):
    mesh = plsc.VectorSubcoreMesh(core_axis_name="c", subcore_axis_name="s")
    out_t = jax.ShapeDtypeStruct((F, B, DW), jnp.int32)

    @functools.partial(
        pl.kernel,
        mesh=mesh,
        out_type=out_t,
        scratch_types=[
            pltpu.VMEM((CH,), jnp.int32),
            pltpu.VMEM((CH,), jnp.int32),
            pltpu.VMEM((CH, DW), jnp.int32),
            pltpu.VMEM((CH, DW), jnp.int32),
            pltpu.SemaphoreType.DMA,
            pltpu.SemaphoreType.DMA,
        ],
        compiler_params=pltpu.CompilerParams(use_tc_tiling_on_sc=True),
    )
    def k(idx_hbm, tab_hbm, out_hbm, i0_v, i1_v, r0, r1, sem0, sem1):
        wid = lax.axis_index("s") * NC + lax.axis_index("c")
        chunks_per_plane = B // CH
        base = wid * NCHUNK

        def out_slice(cg):
            fi = cg // chunks_per_plane
            bs = (cg % chunks_per_plane) * CH
            return out_hbm.at[fi].at[pl.ds(bs, CH)]

        pltpu.sync_copy(idx_hbm.at[pl.ds(base * CH, CH)], i0_v)
        pltpu.async_copy(tab_hbm.at[i0_v], r0, sem0)

        @pl.loop(0, NCHUNK // 2)
        def _(j):
            c0 = base + 2 * j
            pltpu.sync_copy(idx_hbm.at[pl.ds((c0 + 1) * CH, CH)], i1_v)
            pltpu.async_copy(tab_hbm.at[i1_v], r1, sem1)
            pltpu.make_async_copy(tab_hbm.at[i0_v], r0, sem0).wait()
            pltpu.sync_copy(r0, out_slice(c0))

            @pl.when(j < NCHUNK // 2 - 1)
            def _():
                pltpu.sync_copy(idx_hbm.at[pl.ds((c0 + 2) * CH, CH)], i0_v)
                pltpu.async_copy(tab_hbm.at[i0_v], r0, sem0)

            pltpu.make_async_copy(tab_hbm.at[i1_v], r1, sem1).wait()
            pltpu.sync_copy(r1, out_slice(c0 + 1))

    return k(idx, tabp)


def _dense_body(g_ref, w1_ref, b1_ref, w2_ref, b2_ref,
                w3_ref, b3_ref, tw1_ref, tb1_ref, tw2_ref, tb2_ref, out_ref):
    gcat = jnp.concatenate([g_ref[f] for f in range(F)], axis=1)
    logits = []
    for i in range(T):
        v = (gcat << 16) if i == 0 else (gcat & jnp.int32(-65536))
        cc = lax.bitcast_convert_type(v, jnp.float32).astype(jnp.bfloat16)
        h = jnp.maximum(jnp.dot(cc, w1_ref[...], preferred_element_type=jnp.float32) + b1_ref[...], 0.0)
        h = jnp.maximum(jnp.dot(h.astype(jnp.bfloat16), w2_ref[...], preferred_element_type=jnp.float32) + b2_ref[...], 0.0)
        h = jnp.maximum(jnp.dot(h.astype(jnp.bfloat16), w3_ref[...], preferred_element_type=jnp.float32) + b3_ref[...], 0.0)
        t = jnp.maximum(jnp.dot(h.astype(jnp.bfloat16), tw1_ref[i], preferred_element_type=jnp.float32) + tb1_ref[i:i + 1, :], 0.0)
        logit = jnp.sum(t * tw2_ref[i:i + 1, :], axis=1, keepdims=True) + tb2_ref[i:i + 1, :]
        logits.append(logit)
    out_ref[...] = jax.nn.sigmoid(jnp.concatenate(logits, axis=1))


def _tc_dense(g, W1p, b1, W2, b2, W3, b3, tw1, tb1, tw2r, tb2):
    full = lambda shape: pl.BlockSpec(shape, lambda i: (0,) * len(shape))
    return pl.pallas_call(
        _dense_body,
        grid=(B // TB,),
        in_specs=[
            pl.BlockSpec((F, TB, DW), lambda i: (0, i, 0)),
            full((IN_DIM, 256)),
            full((1, 256)),
            full((256, 128)),
            full((1, 128)),
            full((128, 64)),
            full((1, 64)),
            full((T, 64, 64)),
            full((T, 64)),
            full((T, 64)),
            full((T, 1)),
        ],
        out_specs=pl.BlockSpec((TB, T), lambda i: (i, 0)),
        out_shape=jax.ShapeDtypeStruct((B, T), jnp.float32),
    )(g, W1p, b1, W2, b2, W3, b3, tw1, tb1, tw2r, tb2)


def kernel(x, shared_table, task_tables, W1, b1, W2, b2, W3, b3, tw1, tb1, tw2, tb2):
    bf = jnp.bfloat16
    idx = x.T.reshape(BF)
    tab_a = jnp.concatenate([shared_table, task_tables[0]], axis=1).astype(bf)
    tab_b = jnp.concatenate([shared_table, task_tables[1]], axis=1).astype(bf)
    tabp = lax.bitcast_convert_type(jnp.stack([tab_a, tab_b], axis=-1), jnp.int32)
    W1p = W1.reshape(2, F, D, 256).transpose(1, 0, 2, 3).reshape(IN_DIM, 256).astype(bf)
    g = _sc_gather(idx, tabp)
    return _tc_dense(
        g, W1p,
        b1.reshape(1, 256), W2.astype(bf), b2.reshape(1, 128),
        W3.astype(bf), b3.reshape(1, 64),
        tw1.astype(bf), tb1, tw2.reshape(T, 64), tb2,
    )


# speedup vs baseline: 3.7332x; 1.0954x over previous
import functools

import jax
import jax.numpy as jnp
from jax import lax
from jax.experimental import pallas as pl
from jax.experimental.pallas import tpu as pltpu
from jax.experimental.pallas import tpu_sc as plsc

B, F, V, D = 16384, 26, 100000, 64
T = 2
FD = F * D
IN_DIM = 2 * FD
BF = B * F
DW = 2 * D

NC, NS = 2, 16
NW = NC * NS
CH = 128

NSPLIT = 2
BH = B // NSPLIT
NCHUNK = F * BH // CH // NW

TB = 1024


def _sc_gather(idx, tabp):
    mesh = plsc.VectorSubcoreMesh(core_axis_name="c", subcore_axis_name="s")
    out_t = jax.ShapeDtypeStruct((F, BH, DW), jnp.int32)

    @functools.partial(
        pl.kernel,
        mesh=mesh,
        out_type=out_t,
        scratch_types=[
            pltpu.VMEM((CH,), jnp.int32),
            pltpu.VMEM((CH,), jnp.int32),
            pltpu.VMEM((CH, DW), jnp.int32),
            pltpu.VMEM((CH, DW), jnp.int32),
            pltpu.SemaphoreType.DMA,
            pltpu.SemaphoreType.DMA,
        ],
        compiler_params=pltpu.CompilerParams(use_tc_tiling_on_sc=True),
    )
    def k(idx_hbm, tab_hbm, out_hbm, i0_v, i1_v, r0, r1, sem0, sem1):
        wid = lax.axis_index("s") * NC + lax.axis_index("c")
        chunks_per_plane = BH // CH
        base = wid * NCHUNK

        def out_slice(cg):
            fi = cg // chunks_per_plane
            bs = (cg % chunks_per_plane) * CH
            return out_hbm.at[fi].at[pl.ds(bs, CH)]

        pltpu.sync_copy(idx_hbm.at[pl.ds(base * CH, CH)], i0_v)
        pltpu.async_copy(tab_hbm.at[i0_v], r0, sem0)

        @pl.loop(0, NCHUNK // 2)
        def _(j):
            c0 = base + 2 * j
            pltpu.sync_copy(idx_hbm.at[pl.ds((c0 + 1) * CH, CH)], i1_v)
            pltpu.async_copy(tab_hbm.at[i1_v], r1, sem1)
            pltpu.make_async_copy(tab_hbm.at[i0_v], r0, sem0).wait()
            pltpu.sync_copy(r0, out_slice(c0))

            @pl.when(j < NCHUNK // 2 - 1)
            def _():
                pltpu.sync_copy(idx_hbm.at[pl.ds((c0 + 2) * CH, CH)], i0_v)
                pltpu.async_copy(tab_hbm.at[i0_v], r0, sem0)

            pltpu.make_async_copy(tab_hbm.at[i1_v], r1, sem1).wait()
            pltpu.sync_copy(r1, out_slice(c0 + 1))

    return k(idx, tabp)


def _dense_body(g_ref, w1_ref, b1_ref, w2_ref, b2_ref,
                w3_ref, b3_ref, tw1_ref, tb1_ref, tw2_ref, tb2_ref, out_ref):
    gcat = jnp.concatenate([g_ref[f] for f in range(F)], axis=1)
    logits = []
    for i in range(T):
        v = (gcat << 16) if i == 0 else (gcat & jnp.int32(-65536))
        cc = lax.bitcast_convert_type(v, jnp.float32).astype(jnp.bfloat16)
        h = jnp.maximum(jnp.dot(cc, w1_ref[...], preferred_element_type=jnp.float32) + b1_ref[...], 0.0)
        h = jnp.maximum(jnp.dot(h.astype(jnp.bfloat16), w2_ref[...], preferred_element_type=jnp.float32) + b2_ref[...], 0.0)
        h = jnp.maximum(jnp.dot(h.astype(jnp.bfloat16), w3_ref[...], preferred_element_type=jnp.float32) + b3_ref[...], 0.0)
        t = jnp.maximum(jnp.dot(h.astype(jnp.bfloat16), tw1_ref[i], preferred_element_type=jnp.float32) + tb1_ref[i:i + 1, :], 0.0)
        logit = jnp.sum(t * tw2_ref[i:i + 1, :], axis=1, keepdims=True) + tb2_ref[i:i + 1, :]
        logits.append(logit)
    out_ref[...] = jax.nn.sigmoid(jnp.concatenate(logits, axis=1))


def _tc_dense(g, W1p, b1, W2, b2, W3, b3, tw1, tb1, tw2r, tb2):
    full = lambda shape: pl.BlockSpec(shape, lambda i: (0,) * len(shape))
    return pl.pallas_call(
        _dense_body,
        grid=(BH // TB,),
        in_specs=[
            pl.BlockSpec((F, TB, DW), lambda i: (0, i, 0)),
            full((IN_DIM, 256)),
            full((1, 256)),
            full((256, 128)),
            full((1, 128)),
            full((128, 64)),
            full((1, 64)),
            full((T, 64, 64)),
            full((T, 64)),
            full((T, 64)),
            full((T, 1)),
        ],
        out_specs=pl.BlockSpec((TB, T), lambda i: (i, 0)),
        out_shape=jax.ShapeDtypeStruct((BH, T), jnp.float32),
    )(g, W1p, b1, W2, b2, W3, b3, tw1, tb1, tw2r, tb2)


def kernel(x, shared_table, task_tables, W1, b1, W2, b2, W3, b3, tw1, tb1, tw2, tb2):
    bf = jnp.bfloat16
    tab_a = jnp.concatenate([shared_table, task_tables[0]], axis=1).astype(bf)
    tab_b = jnp.concatenate([shared_table, task_tables[1]], axis=1).astype(bf)
    tabp = lax.bitcast_convert_type(jnp.stack([tab_a, tab_b], axis=-1), jnp.int32)
    W1p = W1.reshape(2, F, D, 256).transpose(1, 0, 2, 3).reshape(IN_DIM, 256).astype(bf)
    args = (b1.reshape(1, 256), W2.astype(bf), b2.reshape(1, 128),
            W3.astype(bf), b3.reshape(1, 64),
            tw1.astype(bf), tb1, tw2.reshape(T, 64), tb2)
    outs = []
    for h in range(NSPLIT):
        idx_h = x[h * BH:(h + 1) * BH].T.reshape(F * BH)
        g = _sc_gather(idx_h, tabp)
        outs.append(_tc_dense(g, W1p, *args))
    return jnp.concatenate(outs, axis=0)


# speedup vs baseline: 3.8906x; 1.0422x over previous
import functools

import jax
import jax.numpy as jnp
from jax import lax
from jax.experimental import pallas as pl
from jax.experimental.pallas import tpu as pltpu
from jax.experimental.pallas import tpu_sc as plsc

B, F, V, D = 16384, 26, 100000, 64
T = 2
FD = F * D
IN_DIM = 2 * FD
BF = B * F
DW = 2 * D

NC, NS = 2, 16
NW = NC * NS
CH = 128

NSPLIT = 4
BH = B // NSPLIT
NCHUNK = F * BH // CH // NW

TB = 1024


def _sc_gather(idx, tabp):
    mesh = plsc.VectorSubcoreMesh(core_axis_name="c", subcore_axis_name="s")
    out_t = jax.ShapeDtypeStruct((F, BH, DW), jnp.int32)

    @functools.partial(
        pl.kernel,
        mesh=mesh,
        out_type=out_t,
        scratch_types=[
            pltpu.VMEM((CH,), jnp.int32),
            pltpu.VMEM((CH,), jnp.int32),
            pltpu.VMEM((CH, DW), jnp.int32),
            pltpu.VMEM((CH, DW), jnp.int32),
            pltpu.SemaphoreType.DMA,
            pltpu.SemaphoreType.DMA,
        ],
        compiler_params=pltpu.CompilerParams(use_tc_tiling_on_sc=True),
    )
    def k(idx_hbm, tab_hbm, out_hbm, i0_v, i1_v, r0, r1, sem0, sem1):
        wid = lax.axis_index("s") * NC + lax.axis_index("c")
        chunks_per_plane = BH // CH
        base = wid * NCHUNK

        def out_slice(cg):
            fi = cg // chunks_per_plane
            bs = (cg % chunks_per_plane) * CH
            return out_hbm.at[fi].at[pl.ds(bs, CH)]

        pltpu.sync_copy(idx_hbm.at[pl.ds(base * CH, CH)], i0_v)
        pltpu.async_copy(tab_hbm.at[i0_v], r0, sem0)

        @pl.loop(0, NCHUNK // 2)
        def _(j):
            c0 = base + 2 * j
            pltpu.sync_copy(idx_hbm.at[pl.ds((c0 + 1) * CH, CH)], i1_v)
            pltpu.async_copy(tab_hbm.at[i1_v], r1, sem1)
            pltpu.make_async_copy(tab_hbm.at[i0_v], r0, sem0).wait()
            pltpu.sync_copy(r0, out_slice(c0))

            @pl.when(j < NCHUNK // 2 - 1)
            def _():
                pltpu.sync_copy(idx_hbm.at[pl.ds((c0 + 2) * CH, CH)], i0_v)
                pltpu.async_copy(tab_hbm.at[i0_v], r0, sem0)

            pltpu.make_async_copy(tab_hbm.at[i1_v], r1, sem1).wait()
            pltpu.sync_copy(r1, out_slice(c0 + 1))

    return k(idx, tabp)


def _dense_body(g_ref, w1_ref, b1_ref, w2_ref, b2_ref,
                w3_ref, b3_ref, tw1_ref, tb1_ref, tw2_ref, tb2_ref, out_ref):
    gcat = jnp.concatenate([g_ref[f] for f in range(F)], axis=1)
    logits = []
    for i in range(T):
        v = (gcat << 16) if i == 0 else (gcat & jnp.int32(-65536))
        cc = lax.bitcast_convert_type(v, jnp.float32).astype(jnp.bfloat16)
        h = jnp.maximum(jnp.dot(cc, w1_ref[...], preferred_element_type=jnp.float32) + b1_ref[...], 0.0)
        h = jnp.maximum(jnp.dot(h.astype(jnp.bfloat16), w2_ref[...], preferred_element_type=jnp.float32) + b2_ref[...], 0.0)
        h = jnp.maximum(jnp.dot(h.astype(jnp.bfloat16), w3_ref[...], preferred_element_type=jnp.float32) + b3_ref[...], 0.0)
        t = jnp.maximum(jnp.dot(h.astype(jnp.bfloat16), tw1_ref[i], preferred_element_type=jnp.float32) + tb1_ref[i:i + 1, :], 0.0)
        logit = jnp.sum(t * tw2_ref[i:i + 1, :], axis=1, keepdims=True) + tb2_ref[i:i + 1, :]
        logits.append(logit)
    out_ref[...] = jax.nn.sigmoid(jnp.concatenate(logits, axis=1))


def _tc_dense(g, W1p, b1, W2, b2, W3, b3, tw1, tb1, tw2r, tb2):
    full = lambda shape: pl.BlockSpec(shape, lambda i: (0,) * len(shape))
    return pl.pallas_call(
        _dense_body,
        grid=(BH // TB,),
        in_specs=[
            pl.BlockSpec((F, TB, DW), lambda i: (0, i, 0)),
            full((IN_DIM, 256)),
            full((1, 256)),
            full((256, 128)),
            full((1, 128)),
            full((128, 64)),
            full((1, 64)),
            full((T, 64, 64)),
            full((T, 64)),
            full((T, 64)),
            full((T, 1)),
        ],
        out_specs=pl.BlockSpec((TB, T), lambda i: (i, 0)),
        out_shape=jax.ShapeDtypeStruct((BH, T), jnp.float32),
    )(g, W1p, b1, W2, b2, W3, b3, tw1, tb1, tw2r, tb2)


def kernel(x, shared_table, task_tables, W1, b1, W2, b2, W3, b3, tw1, tb1, tw2, tb2):
    bf = jnp.bfloat16
    tab_a = jnp.concatenate([shared_table, task_tables[0]], axis=1).astype(bf)
    tab_b = jnp.concatenate([shared_table, task_tables[1]], axis=1).astype(bf)
    tabp = lax.bitcast_convert_type(jnp.stack([tab_a, tab_b], axis=-1), jnp.int32)
    W1p = W1.reshape(2, F, D, 256).transpose(1, 0, 2, 3).reshape(IN_DIM, 256).astype(bf)
    args = (b1.reshape(1, 256), W2.astype(bf), b2.reshape(1, 128),
            W3.astype(bf), b3.reshape(1, 64),
            tw1.astype(bf), tb1, tw2.reshape(T, 64), tb2)
    outs = []
    for h in range(NSPLIT):
        idx_h = x[h * BH:(h + 1) * BH].T.reshape(F * BH)
        g = _sc_gather(idx_h, tabp)
        outs.append(_tc_dense(g, W1p, *args))
    return jnp.concatenate(outs, axis=0)
